# baseline (speedup 1.0000x reference)
import functools

import jax
import jax.numpy as jnp
from jax import lax
from jax.experimental import pallas as pl
from jax.experimental.pallas import tpu as pltpu
from jax.experimental.pallas import tpu_sc as plsc

N_NODES = 10000
EMB_DIM = 128
HID = 128
N_LABELS = 512

NC = 2
NS = 16
NW = NC * NS
LANES = 16

N_PAD = 10240
NODE_PER_W = N_PAD // NW
NODE_PER_T = N_PAD // NS

EK = 128
ROWS_PER_W = 88
E_PAD = NW * ROWS_PER_W * EK
NB_E = E_PAD // EK


def _mesh():
    return plsc.VectorSubcoreMesh(core_axis_name="c", subcore_axis_name="s")


def _worker_id():
    return lax.axis_index("c") * NS + lax.axis_index("s")


def _splat_i32(val):
    return jnp.full((LANES,), val, dtype=jnp.int32)


def _deg_body(dst_hbm, ew_hbm, degp_hbm, idx_v, ew_v, z_v, acc, sem):
    del sem
    c = lax.axis_index("c")
    s = lax.axis_index("s")
    wk = c * NS + s

    for i in range(NODE_PER_T // LANES):
        z_v[pl.ds(i * LANES, LANES)] = jnp.zeros((LANES,), jnp.float32)
    pltpu.sync_copy(z_v, acc.at[pl.ds(s * NODE_PER_T, NODE_PER_T)])
    plsc.subcore_barrier()

    base_r = wk * ROWS_PER_W
    pltpu.sync_copy(dst_hbm.at[pl.ds(base_r, ROWS_PER_W)], idx_v)
    pltpu.sync_copy(ew_hbm.at[pl.ds(base_r, ROWS_PER_W)], ew_v)

    def chunk(t, carry):
        pltpu.sync_copy(ew_v.at[t], acc.at[idx_v.at[t]], add=True)
        return carry

    lax.fori_loop(0, ROWS_PER_W, chunk, 0)
    plsc.subcore_barrier()

    pltpu.sync_copy(
        acc.at[pl.ds(s * NODE_PER_T, NODE_PER_T)],
        degp_hbm.at[c, pl.ds(s * NODE_PER_T, NODE_PER_T)],
    )


def _deg_call(dst2d, ew2d):
    return pl.kernel(
        _deg_body,
        out_type=jax.ShapeDtypeStruct((NC, N_PAD), jnp.float32),
        mesh=_mesh(),
        scratch_types=[
            pltpu.VMEM((ROWS_PER_W, EK), jnp.int32),
            pltpu.VMEM((ROWS_PER_W, EK), jnp.float32),
            pltpu.VMEM((NODE_PER_T,), jnp.float32),
            pltpu.VMEM_SHARED((N_PAD,), jnp.float32),
            pltpu.SemaphoreType.DMA,
        ],
    )(dst2d, ew2d)


def _norm_body(src_hbm, dst_hbm, ew_hbm, dis_hbm, w_hbm,
               s_v, d_v, e_v, w_v, dis_v, sem):
    del sem
    wk = _worker_id()
    base_r = wk * ROWS_PER_W

    pltpu.sync_copy(dis_hbm, dis_v)
    pltpu.sync_copy(src_hbm.at[pl.ds(base_r, ROWS_PER_W)], s_v)
    pltpu.sync_copy(dst_hbm.at[pl.ds(base_r, ROWS_PER_W)], d_v)
    pltpu.sync_copy(ew_hbm.at[pl.ds(base_r, ROWS_PER_W)], e_v)

    def row(t, carry):
        for j in range(EK // LANES):
            sl = pl.ds(j * LANES, LANES)
            a = plsc.load_gather(dis_v, [s_v[t, sl]])
            b = plsc.load_gather(dis_v, [d_v[t, sl]])
            w_v[t, sl] = a * e_v[t, sl] * b
        return carry

    lax.fori_loop(0, ROWS_PER_W, row, 0)
    pltpu.sync_copy(w_v, w_hbm.at[pl.ds(base_r, ROWS_PER_W)])


def _norm_call(src2d, dst2d, ew2d, dis):
    return pl.kernel(
        _norm_body,
        out_type=jax.ShapeDtypeStruct((NB_E, EK), jnp.float32),
        mesh=_mesh(),
        compiler_params=pltpu.CompilerParams(needs_layout_passes=False),
        scratch_types=[
            pltpu.VMEM((ROWS_PER_W, EK), jnp.int32),
            pltpu.VMEM((ROWS_PER_W, EK), jnp.int32),
            pltpu.VMEM((ROWS_PER_W, EK), jnp.float32),
            pltpu.VMEM((ROWS_PER_W, EK), jnp.float32),
            pltpu.VMEM((N_PAD,), jnp.float32),
            pltpu.SemaphoreType.DMA,
        ],
    )(src2d, dst2d, ew2d, dis)


_GCHUNK = 64


def _emb_body(nidx_hbm, emb_hbm, x_hbm, idx_v, rows_v, sem):
    wk = _worker_id()
    base = wk * NODE_PER_W
    for t in range(NODE_PER_W // _GCHUNK):
        off = base + t * _GCHUNK
        pltpu.sync_copy(nidx_hbm.at[pl.ds(off, _GCHUNK)], idx_v)
        pltpu.async_copy(emb_hbm.at[idx_v], rows_v, sem).wait()
        pltpu.sync_copy(rows_v, x_hbm.at[pl.ds(off, _GCHUNK), :])


def _emb_call(nidx_pad, emb_table):
    return pl.kernel(
        _emb_body,
        out_type=jax.ShapeDtypeStruct((N_PAD, EMB_DIM), jnp.float32),
        mesh=_mesh(),
        scratch_types=[
            pltpu.VMEM((_GCHUNK,), jnp.int32),
            pltpu.VMEM((_GCHUNK, EMB_DIM), jnp.float32),
            pltpu.SemaphoreType.DMA,
        ],
    )(nidx_pad, emb_table)


_ST = 8
_NST = ROWS_PER_W // _ST


def _spmm_body(src_hbm, dst_hbm, w_hbm, h_hbm, out_hbm,
               si, di, wv, rows0, rows1, acc,
               gsem0, gsem1, ssem0, ssem1):
    c = lax.axis_index("c")
    s = lax.axis_index("s")
    wk = c * NS + s
    base_r = wk * ROWS_PER_W

    def zrow(r, carry):
        for f in range(EMB_DIM // LANES):
            rows0[r, pl.ds(f * LANES, LANES)] = jnp.zeros((LANES,), jnp.float32)
        return carry

    lax.fori_loop(0, EK, zrow, 0)
    for q in range(NODE_PER_T // EK):
        pltpu.sync_copy(rows0, acc.at[pl.ds(s * NODE_PER_T + q * EK, EK), :])
    plsc.subcore_barrier()

    rowbufs = (rows0, rows1)
    gsems = (gsem0, gsem1)
    ssems = (ssem0, ssem1)

    def scale(rows, t):
        @plsc.parallel_loop(0, EK // LANES, unroll=4)
        def group(g):
            for i16 in range(LANES):
                i = g * LANES + i16
                bw = plsc.load_gather(wv, [_splat_i32(t), _splat_i32(i)])
                for f in range(EMB_DIM // LANES):
                    sl = pl.ds(f * LANES, LANES)
                    rows[i, sl] = rows[i, sl] * bw

    def stage(st, carry):
        r0 = base_r + st * _ST
        pltpu.sync_copy(src_hbm.at[pl.ds(r0, _ST)], si)
        pltpu.sync_copy(dst_hbm.at[pl.ds(r0, _ST)], di)
        pltpu.sync_copy(w_hbm.at[pl.ds(r0, _ST)], wv)

        gds = [None] * _ST
        sds = [None] * _ST
        gds[0] = pltpu.async_copy(h_hbm.at[si.at[0]], rows0, gsem0)
        for t in range(_ST):
            b = t % 2
            gds[t].wait()
            scale(rowbufs[b], t)
            if t + 1 < _ST:
                if t - 1 >= 0:
                    sds[t - 1].wait()
                gds[t + 1] = pltpu.async_copy(
                    h_hbm.at[si.at[t + 1]], rowbufs[1 - b], gsems[1 - b])
            sds[t] = pltpu.async_copy(
                rowbufs[b], acc.at[di.at[t]], ssems[b], add=True)
        sds[_ST - 2].wait()
        sds[_ST - 1].wait()
        return carry

    lax.fori_loop(0, _NST, stage, 0)
    plsc.subcore_barrier()

    for q in range(NODE_PER_T // EK):
        r0 = s * NODE_PER_T + q * EK
        pltpu.sync_copy(acc.at[pl.ds(r0, EK), :], out_hbm.at[c, pl.ds(r0, EK), :])


def _spmm_call(src2d, dst2d, w2d, h):
    return pl.kernel(
        _spmm_body,
        out_type=jax.ShapeDtypeStruct((NC, N_PAD, HID), jnp.float32),
        mesh=_mesh(),
        compiler_params=pltpu.CompilerParams(needs_layout_passes=False),
        scratch_types=[
            pltpu.VMEM((_ST, EK), jnp.int32),
            pltpu.VMEM((_ST, EK), jnp.int32),
            pltpu.VMEM((_ST, EK), jnp.float32),
            pltpu.VMEM((EK, HID), jnp.float32),
            pltpu.VMEM((EK, HID), jnp.float32),
            pltpu.VMEM_SHARED((N_PAD, HID), jnp.float32),
            pltpu.SemaphoreType.DMA,
            pltpu.SemaphoreType.DMA,
            pltpu.SemaphoreType.DMA,
            pltpu.SemaphoreType.DMA,
        ],
    )(src2d, dst2d, w2d, h)


def _dis_body(degp_ref, dis_ref):
    deg = degp_ref[0] + degp_ref[1]
    dis_ref[...] = jnp.where(deg > 0, lax.rsqrt(deg), 0.0)


def _dis_call(degp):
    degp3 = degp.reshape(NC, N_PAD // 128, 128)
    out = pl.pallas_call(
        _dis_body,
        out_shape=jax.ShapeDtypeStruct((N_PAD // 128, 128), jnp.float32),
    )(degp3)
    return out.reshape(N_PAD)


_RB = 400


def _mm_body(x_ref, w_ref, o_ref):
    o_ref[...] = jnp.dot(x_ref[...], w_ref[...],
                         preferred_element_type=jnp.float32)


def _mm_call(x, w):
    n, k = x.shape
    m = w.shape[1]
    return pl.pallas_call(
        _mm_body,
        grid=(n // _RB,),
        in_specs=[
            pl.BlockSpec((_RB, k), lambda i: (i, 0)),
            pl.BlockSpec((k, m), lambda i: (0, 0)),
        ],
        out_specs=pl.BlockSpec((_RB, m), lambda i: (i, 0)),
        out_shape=jax.ShapeDtypeStruct((n, m), jnp.float32),
    )(x, w)


def _combine_mm_body(s0_ref, s1_ref, b_ref, w_ref, bout_ref, o_ref):
    h = s0_ref[...] + s1_ref[...] + b_ref[0:1, :]
    h = jnp.maximum(h, 0.0)
    o_ref[...] = (jnp.dot(h, w_ref[...], preferred_element_type=jnp.float32)
                  + bout_ref[0:1, :])


def _combine_mm_call(s0, s1, b, w, bout):
    n, k = s0.shape
    m = w.shape[1]
    b2 = jnp.broadcast_to(b.reshape(1, k), (8, k))
    bo2 = jnp.broadcast_to(bout.reshape(1, m), (8, m))
    return pl.pallas_call(
        _combine_mm_body,
        grid=(n // _RB,),
        in_specs=[
            pl.BlockSpec((_RB, k), lambda i: (i, 0)),
            pl.BlockSpec((_RB, k), lambda i: (i, 0)),
            pl.BlockSpec((8, k), lambda i: (0, 0)),
            pl.BlockSpec((k, m), lambda i: (0, 0)),
            pl.BlockSpec((8, m), lambda i: (0, 0)),
        ],
        out_specs=pl.BlockSpec((_RB, m), lambda i: (i, 0)),
        out_shape=jax.ShapeDtypeStruct((n, m), jnp.float32),
    )(s0, s1, b2, w, bo2)


def kernel(node_indices, edge_index, edge_weight, emb_table,
           W1, b1, W2, b2, Wout, bout):
    n_edges = edge_index.shape[1]
    loop = jnp.arange(N_NODES, dtype=jnp.int32)
    pad = E_PAD - (n_edges + N_NODES)

    pad_dst = jnp.arange(pad, dtype=jnp.int32) % N_PAD
    pad_src = jnp.arange(pad, dtype=jnp.int32) % N_NODES
    src_f = jnp.concatenate(
        [edge_index[0], loop, pad_src]).reshape(NB_E, EK)
    dst_f = jnp.concatenate(
        [edge_index[1], loop, pad_dst]).reshape(NB_E, EK)
    ew_f = jnp.concatenate(
        [edge_weight, jnp.ones((N_NODES,), jnp.float32),
         jnp.zeros((pad,), jnp.float32)]).reshape(NB_E, EK)

    nidx_pad = jnp.concatenate(
        [node_indices.astype(jnp.int32),
         jnp.zeros((N_PAD - N_NODES,), jnp.int32)])

    degp = _deg_call(dst_f, ew_f)
    dis = _dis_call(degp)
    w2d = _norm_call(src_f, dst_f, ew_f, dis)

    x = _emb_call(nidx_pad, emb_table)[:N_NODES]

    h1 = _mm_call(x, W1)
    S = _spmm_call(src_f, dst_f, w2d, h1)
    h2 = _combine_mm_call(S[0, :N_NODES], S[1, :N_NODES], b1, W2,
                          jnp.zeros((HID,), jnp.float32))

    S2 = _spmm_call(src_f, dst_f, w2d, h2)
    logits = _combine_mm_call(S2[0, :N_NODES], S2[1, :N_NODES], b2, Wout, bout)
    return logits


# speedup vs baseline: 1.1535x; 1.1535x over previous
import functools

import jax
import jax.numpy as jnp
from jax import lax
from jax.experimental import pallas as pl
from jax.experimental.pallas import tpu as pltpu
from jax.experimental.pallas import tpu_sc as plsc

N_NODES = 10000
EMB_DIM = 128
HID = 128
N_LABELS = 512

NC = 2
NS = 16
NW = NC * NS
LANES = 16

N_PAD = 10240
NODE_PER_W = N_PAD // NW
NODE_PER_T = N_PAD // NS

EK = 128
ROWS_PER_W = 88
E_PAD = NW * ROWS_PER_W * EK
NB_E = E_PAD // EK


def _mesh():
    return plsc.VectorSubcoreMesh(core_axis_name="c", subcore_axis_name="s")


def _worker_id():
    return lax.axis_index("c") * NS + lax.axis_index("s")


def _splat_i32(val):
    return jnp.full((LANES,), val, dtype=jnp.int32)


def _deg_body(dst_hbm, ew_hbm, degp_hbm, idx_v, ew_v, z_v, acc, sem):
    del sem
    c = lax.axis_index("c")
    s = lax.axis_index("s")
    wk = c * NS + s

    for i in range(NODE_PER_T // LANES):
        z_v[pl.ds(i * LANES, LANES)] = jnp.zeros((LANES,), jnp.float32)
    pltpu.sync_copy(z_v, acc.at[pl.ds(s * NODE_PER_T, NODE_PER_T)])
    plsc.subcore_barrier()

    base_r = wk * ROWS_PER_W
    pltpu.sync_copy(dst_hbm.at[pl.ds(base_r, ROWS_PER_W)], idx_v)
    pltpu.sync_copy(ew_hbm.at[pl.ds(base_r, ROWS_PER_W)], ew_v)

    def chunk(t, carry):
        pltpu.sync_copy(ew_v.at[t], acc.at[idx_v.at[t]], add=True)
        return carry

    lax.fori_loop(0, ROWS_PER_W, chunk, 0)
    plsc.subcore_barrier()

    pltpu.sync_copy(
        acc.at[pl.ds(s * NODE_PER_T, NODE_PER_T)],
        degp_hbm.at[c, pl.ds(s * NODE_PER_T, NODE_PER_T)],
    )


def _deg_call(dst2d, ew2d):
    return pl.kernel(
        _deg_body,
        out_type=jax.ShapeDtypeStruct((NC, N_PAD), jnp.float32),
        mesh=_mesh(),
        scratch_types=[
            pltpu.VMEM((ROWS_PER_W, EK), jnp.int32),
            pltpu.VMEM((ROWS_PER_W, EK), jnp.float32),
            pltpu.VMEM((NODE_PER_T,), jnp.float32),
            pltpu.VMEM_SHARED((N_PAD,), jnp.float32),
            pltpu.SemaphoreType.DMA,
        ],
    )(dst2d, ew2d)


def _norm_body(src_hbm, dst_hbm, ew_hbm, dis_hbm, w_hbm,
               s_v, d_v, e_v, w_v, dis_v, sem):
    del sem
    wk = _worker_id()
    base_r = wk * ROWS_PER_W

    pltpu.sync_copy(dis_hbm, dis_v)
    pltpu.sync_copy(src_hbm.at[pl.ds(base_r, ROWS_PER_W)], s_v)
    pltpu.sync_copy(dst_hbm.at[pl.ds(base_r, ROWS_PER_W)], d_v)
    pltpu.sync_copy(ew_hbm.at[pl.ds(base_r, ROWS_PER_W)], e_v)

    def row(t, carry):
        for j in range(EK // LANES):
            sl = pl.ds(j * LANES, LANES)
            a = plsc.load_gather(dis_v, [s_v[t, sl]])
            b = plsc.load_gather(dis_v, [d_v[t, sl]])
            w_v[t, sl] = a * e_v[t, sl] * b
        return carry

    lax.fori_loop(0, ROWS_PER_W, row, 0)
    pltpu.sync_copy(w_v, w_hbm.at[pl.ds(base_r, ROWS_PER_W)])


def _norm_call(src2d, dst2d, ew2d, dis):
    return pl.kernel(
        _norm_body,
        out_type=jax.ShapeDtypeStruct((NB_E, EK), jnp.float32),
        mesh=_mesh(),
        compiler_params=pltpu.CompilerParams(needs_layout_passes=False),
        scratch_types=[
            pltpu.VMEM((ROWS_PER_W, EK), jnp.int32),
            pltpu.VMEM((ROWS_PER_W, EK), jnp.int32),
            pltpu.VMEM((ROWS_PER_W, EK), jnp.float32),
            pltpu.VMEM((ROWS_PER_W, EK), jnp.float32),
            pltpu.VMEM((N_PAD,), jnp.float32),
            pltpu.SemaphoreType.DMA,
        ],
    )(src2d, dst2d, ew2d, dis)


_GCHUNK = 64


def _emb_body(nidx_hbm, emb_hbm, x_hbm, idx_v, rows_v, sem):
    wk = _worker_id()
    base = wk * NODE_PER_W
    for t in range(NODE_PER_W // _GCHUNK):
        off = base + t * _GCHUNK
        pltpu.sync_copy(nidx_hbm.at[pl.ds(off, _GCHUNK)], idx_v)
        pltpu.async_copy(emb_hbm.at[idx_v], rows_v, sem).wait()
        pltpu.sync_copy(rows_v, x_hbm.at[pl.ds(off, _GCHUNK), :])


def _emb_call(nidx_pad, emb_table):
    return pl.kernel(
        _emb_body,
        out_type=jax.ShapeDtypeStruct((N_PAD, EMB_DIM), jnp.float32),
        mesh=_mesh(),
        scratch_types=[
            pltpu.VMEM((_GCHUNK,), jnp.int32),
            pltpu.VMEM((_GCHUNK, EMB_DIM), jnp.float32),
            pltpu.SemaphoreType.DMA,
        ],
    )(nidx_pad, emb_table)


_ST = 8
_NST = ROWS_PER_W // _ST


def _spmm_body(src_hbm, dst_hbm, w_hbm, h_hbm, out_hbm,
               si, di, wv, rows0, rows1, acc,
               gsem0, gsem1, ssem0, ssem1):
    c = lax.axis_index("c")
    s = lax.axis_index("s")
    wk = c * NS + s
    base_r = wk * ROWS_PER_W

    def zrow(r, carry):
        for f in range(EMB_DIM // LANES):
            rows0[r, pl.ds(f * LANES, LANES)] = jnp.zeros((LANES,), jnp.float32)
        return carry

    lax.fori_loop(0, EK, zrow, 0)
    for q in range(NODE_PER_T // EK):
        pltpu.sync_copy(rows0, acc.at[pl.ds(s * NODE_PER_T + q * EK, EK), :])
    plsc.subcore_barrier()

    rowbufs = (rows0, rows1)
    gsems = (gsem0, gsem1)
    ssems = (ssem0, ssem1)

    def scale(rows, t):
        @plsc.parallel_loop(0, EK // LANES, unroll=2)
        def group(g):
            for i16 in range(LANES):
                i = g * LANES + i16
                bw = plsc.load_gather(wv, [_splat_i32(t), _splat_i32(i)])
                for f in range(EMB_DIM // LANES):
                    sl = pl.ds(f * LANES, LANES)
                    rows[i, sl] = rows[i, sl] * bw

    def stage(st, carry):
        r0 = base_r + st * _ST
        d1 = pltpu.async_copy(src_hbm.at[pl.ds(r0, _ST)], si, gsem0)
        d2 = pltpu.async_copy(dst_hbm.at[pl.ds(r0, _ST)], di, gsem0)
        d3 = pltpu.async_copy(w_hbm.at[pl.ds(r0, _ST)], wv, gsem0)
        d1.wait()
        d2.wait()
        d3.wait()

        gds = [None] * _ST
        sds = [None] * _ST
        gds[0] = pltpu.async_copy(h_hbm.at[si.at[0]], rows0, gsem0)
        for t in range(_ST):
            b = t % 2
            gds[t].wait()
            scale(rowbufs[b], t)
            if t + 1 < _ST:
                if t - 1 >= 0:
                    sds[t - 1].wait()
                gds[t + 1] = pltpu.async_copy(
                    h_hbm.at[si.at[t + 1]], rowbufs[1 - b], gsems[1 - b])
            sds[t] = pltpu.async_copy(
                rowbufs[b], acc.at[di.at[t]], ssems[b], add=True)
        sds[_ST - 2].wait()
        sds[_ST - 1].wait()
        return carry

    lax.fori_loop(0, _NST, stage, 0)
    plsc.subcore_barrier()

    for q in range(NODE_PER_T // EK):
        r0 = s * NODE_PER_T + q * EK
        pltpu.sync_copy(acc.at[pl.ds(r0, EK), :], out_hbm.at[c, pl.ds(r0, EK), :])


def _spmm_call(src2d, dst2d, w2d, h):
    return pl.kernel(
        _spmm_body,
        out_type=jax.ShapeDtypeStruct((NC, N_PAD, HID), jnp.float32),
        mesh=_mesh(),
        compiler_params=pltpu.CompilerParams(needs_layout_passes=False),
        scratch_types=[
            pltpu.VMEM((_ST, EK), jnp.int32),
            pltpu.VMEM((_ST, EK), jnp.int32),
            pltpu.VMEM((_ST, EK), jnp.float32),
            pltpu.VMEM((EK, HID), jnp.float32),
            pltpu.VMEM((EK, HID), jnp.float32),
            pltpu.VMEM_SHARED((N_PAD, HID), jnp.float32),
            pltpu.SemaphoreType.DMA,
            pltpu.SemaphoreType.DMA,
            pltpu.SemaphoreType.DMA,
            pltpu.SemaphoreType.DMA,
        ],
    )(src2d, dst2d, w2d, h)


def _dis_body(degp_ref, dis_ref):
    deg = degp_ref[0] + degp_ref[1]
    dis_ref[...] = jnp.where(deg > 0, lax.rsqrt(deg), 0.0)


def _dis_call(degp):
    degp3 = degp.reshape(NC, N_PAD // 128, 128)
    out = pl.pallas_call(
        _dis_body,
        out_shape=jax.ShapeDtypeStruct((N_PAD // 128, 128), jnp.float32),
    )(degp3)
    return out.reshape(N_PAD)


_RB = 400


def _mm_body(x_ref, w_ref, o_ref):
    o_ref[...] = jnp.dot(x_ref[...], w_ref[...],
                         preferred_element_type=jnp.float32)


def _mm_call(x, w):
    n, k = x.shape
    m = w.shape[1]
    return pl.pallas_call(
        _mm_body,
        grid=(n // _RB,),
        in_specs=[
            pl.BlockSpec((_RB, k), lambda i: (i, 0)),
            pl.BlockSpec((k, m), lambda i: (0, 0)),
        ],
        out_specs=pl.BlockSpec((_RB, m), lambda i: (i, 0)),
        out_shape=jax.ShapeDtypeStruct((n, m), jnp.float32),
    )(x, w)


def _combine_mm_body(s0_ref, s1_ref, b_ref, w_ref, bout_ref, o_ref):
    h = s0_ref[...] + s1_ref[...] + b_ref[0:1, :]
    h = jnp.maximum(h, 0.0)
    o_ref[...] = (jnp.dot(h, w_ref[...], preferred_element_type=jnp.float32)
                  + bout_ref[0:1, :])


def _combine_mm_call(s0, s1, b, w, bout):
    n, k = s0.shape
    m = w.shape[1]
    b2 = jnp.broadcast_to(b.reshape(1, k), (8, k))
    bo2 = jnp.broadcast_to(bout.reshape(1, m), (8, m))
    return pl.pallas_call(
        _combine_mm_body,
        grid=(n // _RB,),
        in_specs=[
            pl.BlockSpec((_RB, k), lambda i: (i, 0)),
            pl.BlockSpec((_RB, k), lambda i: (i, 0)),
            pl.BlockSpec((8, k), lambda i: (0, 0)),
            pl.BlockSpec((k, m), lambda i: (0, 0)),
            pl.BlockSpec((8, m), lambda i: (0, 0)),
        ],
        out_specs=pl.BlockSpec((_RB, m), lambda i: (i, 0)),
        out_shape=jax.ShapeDtypeStruct((n, m), jnp.float32),
    )(s0, s1, b2, w, bo2)


def kernel(node_indices, edge_index, edge_weight, emb_table,
           W1, b1, W2, b2, Wout, bout):
    n_edges = edge_index.shape[1]
    loop = jnp.arange(N_NODES, dtype=jnp.int32)
    pad = E_PAD - (n_edges + N_NODES)

    pad_dst = jnp.arange(pad, dtype=jnp.int32) % N_PAD
    pad_src = jnp.arange(pad, dtype=jnp.int32) % N_NODES
    src_f = jnp.concatenate(
        [edge_index[0], loop, pad_src]).reshape(NB_E, EK)
    dst_f = jnp.concatenate(
        [edge_index[1], loop, pad_dst]).reshape(NB_E, EK)
    ew_f = jnp.concatenate(
        [edge_weight, jnp.ones((N_NODES,), jnp.float32),
         jnp.zeros((pad,), jnp.float32)]).reshape(NB_E, EK)

    nidx_pad = jnp.concatenate(
        [node_indices.astype(jnp.int32),
         jnp.zeros((N_PAD - N_NODES,), jnp.int32)])

    degp = _deg_call(dst_f, ew_f)
    dis = _dis_call(degp)
    w2d = _norm_call(src_f, dst_f, ew_f, dis)

    x = _emb_call(nidx_pad, emb_table)[:N_NODES]

    h1 = _mm_call(x, W1)
    S = _spmm_call(src_f, dst_f, w2d, h1)
    h2 = _combine_mm_call(S[0, :N_NODES], S[1, :N_NODES], b1, W2,
                          jnp.zeros((HID,), jnp.float32))

    S2 = _spmm_call(src_f, dst_f, w2d, h2)
    logits = _combine_mm_call(S2[0, :N_NODES], S2[1, :N_NODES], b2, Wout, bout)
    return logits
